# baseline (speedup 1.0000x reference)
import functools

import jax
import jax.numpy as jnp
from jax import lax
from jax.experimental import pallas as pl
from jax.experimental.pallas import tpu as pltpu
from jax.experimental.pallas import tpu_sc as plsc

SEQ = 4
NAA = 20
VOCAB = NAA**SEQ
B = 16384
F = SEQ * NAA
NC, NS, L = 2, 16, 16
NW = NC * NS
BPW = B // NW
CHUNK = 128
NCHUNK = BPW // CHUNK
NGRP = CHUNK // L
NACC = 8

_mesh = plsc.VectorSubcoreMesh(
    core_axis_name="c", subcore_axis_name="s", num_cores=NC, num_subcores=NS
)


@functools.partial(
    pl.kernel,
    out_type=jax.ShapeDtypeStruct((B,), jnp.float32),
    mesh=_mesh,
    compiler_params=pltpu.CompilerParams(needs_layout_passes=False),
    scratch_types=[
        pltpu.VMEM((CHUNK * F,), jnp.float32),
        pltpu.VMEM((CHUNK * F,), jnp.float32),
        pltpu.VMEM((F,), jnp.float32),
        pltpu.VMEM((BPW,), jnp.int32),
        pltpu.VMEM((BPW,), jnp.float32),
        pltpu.SemaphoreType.DMA,
        pltpu.SemaphoreType.DMA,
        pltpu.SemaphoreType.DMA,
    ],
)
def _sc_fwd(x_hbm, fit_hbm, mf_hbm, out_hbm, x_v0, x_v1, mf_v, idx_v, val_v, s0, s1, sg):
    wid = lax.axis_index("s") * NC + lax.axis_index("c")
    base = wid * BPW

    pltpu.sync_copy(mf_hbm, mf_v)
    mf_blocks = [mf_v[pl.ds(k * L, L)] for k in range(F // L)]

    lanes = lax.iota(jnp.int32, L)
    rot = [jnp.bitwise_and(lanes + t, L - 1) for t in range(L)]
    xbufs = (x_v0, x_v1)
    xsem = (s0, s1)

    def start_fetch(c):
        return pltpu.async_copy(
            x_hbm.at[pl.ds((base + c * CHUNK) * F, CHUNK * F)],
            xbufs[c % 2],
            xsem[c % 2],
        )

    pending_x = start_fetch(0)
    gathers = []
    for c in range(NCHUNK):
        nxt = start_fetch(c + 1) if c + 1 < NCHUNK else None
        pending_x.wait()
        pending_x = nxt
        xbuf = xbufs[c % 2]

        def group_body(g, carry, _c=c, _xbuf=xbuf):
            row0 = pl.multiple_of(g * L, L)
            flat0 = (row0 + lanes) * F
            accs = [jnp.zeros((L,), jnp.float32) for _ in range(NACC)]
            for b in range(F // L):
                flat_b = flat0 + b * L
                for t in range(L):
                    v = plsc.load_gather(_xbuf, [flat_b + rot[t]])
                    w = mf_blocks[b].at[rot[t]].get(mode="promise_in_bounds")
                    a = (b * L + t) % NACC
                    accs[a] = accs[a] + v * w
            while len(accs) > 1:
                accs = [
                    accs[i] + accs[i + 1] if i + 1 < len(accs) else accs[i]
                    for i in range(0, len(accs), 2)
                ]
            idx16 = jnp.clip(accs[0], 0.0, float(VOCAB - 1)).astype(jnp.int32)
            idx_v[pl.ds(pl.multiple_of(_c * CHUNK + row0, L), L)] = idx16
            return carry

        lax.fori_loop(0, NGRP, group_body, 0)
        gathers.append(
            pltpu.async_copy(
                fit_hbm.at[idx_v.at[pl.ds(c * CHUNK, CHUNK)]],
                val_v.at[pl.ds(c * CHUNK, CHUNK)],
                sg,
            )
        )

    for gcopy in gathers:
        gcopy.wait()
    pltpu.sync_copy(val_v, out_hbm.at[pl.ds(base, BPW)])


def kernel(x, fitnesses, mult_factor):
    x_flat = x.reshape(B * F)
    fit_flat = fitnesses.reshape(VOCAB)
    mf_flat = mult_factor.reshape(F)
    return _sc_fwd(x_flat, fit_flat, mf_flat)


# speedup vs baseline: 1.9494x; 1.9494x over previous
import functools

import jax
import jax.numpy as jnp
from jax import lax
from jax.experimental import pallas as pl
from jax.experimental.pallas import tpu as pltpu
from jax.experimental.pallas import tpu_sc as plsc

SEQ = 4
NAA = 20
VOCAB = NAA**SEQ
B = 16384
NC, NS, L = 2, 16, 16
NW = NC * NS
BPW = B // NW
GCHUNK = 128
NGATHER = BPW // GCHUNK
TCB = 512

_mesh = plsc.VectorSubcoreMesh(
    core_axis_name="c", subcore_axis_name="s", num_cores=NC, num_subcores=NS
)


def _tc_index_body(x_ref, mf_ref, o_ref):
    xb = x_ref[...]
    s = jnp.sum(xb * mf_ref[...][None], axis=(1, 2))
    o_ref[...] = jnp.clip(s, 0.0, float(VOCAB - 1)).astype(jnp.int32)


_tc_index = pl.pallas_call(
    _tc_index_body,
    grid=(B // TCB,),
    in_specs=[
        pl.BlockSpec((TCB, SEQ, NAA), lambda i: (i, 0, 0)),
        pl.BlockSpec((SEQ, NAA), lambda i: (0, 0)),
    ],
    out_specs=pl.BlockSpec((TCB,), lambda i: (i,)),
    out_shape=jax.ShapeDtypeStruct((B,), jnp.int32),
)


@functools.partial(
    pl.kernel,
    out_type=jax.ShapeDtypeStruct((B,), jnp.float32),
    mesh=_mesh,
    compiler_params=pltpu.CompilerParams(needs_layout_passes=False),
    scratch_types=[
        pltpu.VMEM_SHARED((VOCAB,), jnp.float32),
        pltpu.VMEM((BPW,), jnp.int32),
        pltpu.VMEM((BPW,), jnp.float32),
        pltpu.SemaphoreType.DMA,
        pltpu.SemaphoreType.DMA,
    ],
)
def _sc_gather(idx_hbm, fit_hbm, out_hbm, fit_s, idx_v, val_v, si, sg):
    cid = lax.axis_index("c")
    sid = lax.axis_index("s")
    base = (sid * NC + cid) * BPW

    idx_cp = pltpu.async_copy(idx_hbm.at[pl.ds(base, BPW)], idx_v, si)

    @pl.when(sid == 0)
    def _stage_table():
        pltpu.sync_copy(fit_hbm, fit_s)

    plsc.subcore_barrier()
    idx_cp.wait()

    gathers = [
        pltpu.async_copy(
            fit_s.at[idx_v.at[pl.ds(t * GCHUNK, GCHUNK)]],
            val_v.at[pl.ds(t * GCHUNK, GCHUNK)],
            sg,
        )
        for t in range(NGATHER)
    ]
    for g in gathers:
        g.wait()
    pltpu.sync_copy(val_v, out_hbm.at[pl.ds(base, BPW)])


def kernel(x, fitnesses, mult_factor):
    idx = _tc_index(x, mult_factor)
    return _sc_gather(idx, fitnesses.reshape(VOCAB))


# speedup vs baseline: 2.4647x; 1.2644x over previous
import functools

import jax
import jax.numpy as jnp
from jax import lax
from jax.experimental import pallas as pl
from jax.experimental.pallas import tpu as pltpu
from jax.experimental.pallas import tpu_sc as plsc

SEQ = 4
NAA = 20
VOCAB = NAA**SEQ
B = 16384
NC, NS, L = 2, 16, 16
NW = NC * NS
BPW = B // NW
GCHUNK = 128
NGATHER = BPW // GCHUNK
TCB = 512

_mesh = plsc.VectorSubcoreMesh(
    core_axis_name="c", subcore_axis_name="s", num_cores=NC, num_subcores=NS
)


F = SEQ * NAA


def _tc_index_body(x_ref, mf_ref, o_ref):
    s = jax.lax.dot_general(
        x_ref[...],
        mf_ref[...],
        (((1,), (0,)), ((), ())),
        preferred_element_type=jnp.float32,
    )
    o_ref[...] = jnp.clip(s, 0.0, float(VOCAB - 1)).astype(jnp.int32)


_tc_index = pl.pallas_call(
    _tc_index_body,
    grid=(B // TCB,),
    in_specs=[
        pl.BlockSpec((TCB, F), lambda i: (i, 0)),
        pl.BlockSpec((F, 1), lambda i: (0, 0)),
    ],
    out_specs=pl.BlockSpec((TCB, 1), lambda i: (i, 0)),
    out_shape=jax.ShapeDtypeStruct((B, 1), jnp.int32),
)


@functools.partial(
    pl.kernel,
    out_type=jax.ShapeDtypeStruct((B,), jnp.float32),
    mesh=_mesh,
    compiler_params=pltpu.CompilerParams(needs_layout_passes=False),
    scratch_types=[
        pltpu.VMEM_SHARED((VOCAB,), jnp.float32),
        pltpu.VMEM((BPW,), jnp.int32),
        pltpu.VMEM((BPW,), jnp.float32),
        pltpu.SemaphoreType.DMA,
        pltpu.SemaphoreType.DMA,
    ],
)
def _sc_gather(idx_hbm, fit_hbm, out_hbm, fit_s, idx_v, val_v, si, sg):
    cid = lax.axis_index("c")
    sid = lax.axis_index("s")
    base = (sid * NC + cid) * BPW

    idx_cp = pltpu.async_copy(idx_hbm.at[pl.ds(base, BPW)], idx_v, si)

    @pl.when(sid == 0)
    def _stage_table():
        pltpu.sync_copy(fit_hbm, fit_s)

    plsc.subcore_barrier()
    idx_cp.wait()

    gathers = [
        pltpu.async_copy(
            fit_s.at[idx_v.at[pl.ds(t * GCHUNK, GCHUNK)]],
            val_v.at[pl.ds(t * GCHUNK, GCHUNK)],
            sg,
        )
        for t in range(NGATHER)
    ]
    for g in gathers:
        g.wait()
    pltpu.sync_copy(val_v, out_hbm.at[pl.ds(base, BPW)])


def kernel(x, fitnesses, mult_factor):
    idx = _tc_index(x.reshape(B, F), mult_factor.reshape(F, 1))
    return _sc_gather(idx.reshape(B), fitnesses.reshape(VOCAB))


# speedup vs baseline: 2.5835x; 1.0482x over previous
import functools

import jax
import jax.numpy as jnp
from jax import lax
from jax.experimental import pallas as pl
from jax.experimental.pallas import tpu as pltpu
from jax.experimental.pallas import tpu_sc as plsc

SEQ = 4
NAA = 20
VOCAB = NAA**SEQ
B = 16384
NC, NS, L = 2, 16, 16
NW = NC * NS
BPW = B // NW
GCHUNK = 128
NGATHER = BPW // GCHUNK
TCB = 512

_mesh = plsc.VectorSubcoreMesh(
    core_axis_name="c", subcore_axis_name="s", num_cores=NC, num_subcores=NS
)


F = SEQ * NAA


def _tc_index_body(x_ref, mf_ref, o_ref):
    s = jax.lax.dot_general(
        x_ref[...],
        mf_ref[...],
        (((1,), (0,)), ((), ())),
        preferred_element_type=jnp.float32,
    )
    idx = jnp.clip(s, 0.0, float(VOCAB - 1)).astype(jnp.int32)
    o_ref[...] = idx.reshape(TCB)


_tc_index = pl.pallas_call(
    _tc_index_body,
    grid=(B // TCB,),
    in_specs=[
        pl.BlockSpec((TCB, F), lambda i: (i, 0)),
        pl.BlockSpec((F, 1), lambda i: (0, 0)),
    ],
    out_specs=pl.BlockSpec((TCB,), lambda i: (i,)),
    out_shape=jax.ShapeDtypeStruct((B,), jnp.int32),
)


@functools.partial(
    pl.kernel,
    out_type=jax.ShapeDtypeStruct((B,), jnp.float32),
    mesh=_mesh,
    compiler_params=pltpu.CompilerParams(needs_layout_passes=False),
    scratch_types=[
        pltpu.VMEM_SHARED((VOCAB,), jnp.float32),
        pltpu.VMEM((BPW,), jnp.int32),
        pltpu.VMEM((BPW,), jnp.float32),
        pltpu.SemaphoreType.DMA,
        pltpu.SemaphoreType.DMA,
    ],
)
def _sc_gather(idx_hbm, fit_hbm, out_hbm, fit_s, idx_v, val_v, si, sg):
    cid = lax.axis_index("c")
    sid = lax.axis_index("s")
    base = (sid * NC + cid) * BPW

    idx_cp = pltpu.async_copy(idx_hbm.at[pl.ds(base, BPW)], idx_v, si)

    @pl.when(sid == 0)
    def _stage_table():
        pltpu.sync_copy(fit_hbm, fit_s)

    plsc.subcore_barrier()
    idx_cp.wait()

    gathers = [
        pltpu.async_copy(
            fit_s.at[idx_v.at[pl.ds(t * GCHUNK, GCHUNK)]],
            val_v.at[pl.ds(t * GCHUNK, GCHUNK)],
            sg,
        )
        for t in range(NGATHER)
    ]
    for g in gathers:
        g.wait()
    pltpu.sync_copy(val_v, out_hbm.at[pl.ds(base, BPW)])


def kernel(x, fitnesses, mult_factor):
    idx = _tc_index(x.reshape(B, F), mult_factor.reshape(F, 1))
    return _sc_gather(idx, fitnesses.reshape(VOCAB))


# speedup vs baseline: 3.1714x; 1.2275x over previous
import functools

import jax
import jax.numpy as jnp
from jax import lax
from jax.experimental import pallas as pl
from jax.experimental.pallas import tpu as pltpu
from jax.experimental.pallas import tpu_sc as plsc

SEQ = 4
NAA = 20
VOCAB = NAA**SEQ
B = 16384
NC, NS, L = 2, 16, 16
NW = NC * NS
BPW = B // NW
GCHUNK = 128
NGATHER = BPW // GCHUNK
TCB = 2048

_mesh = plsc.VectorSubcoreMesh(
    core_axis_name="c", subcore_axis_name="s", num_cores=NC, num_subcores=NS
)


F = SEQ * NAA


def _tc_index_body(x_ref, mf_ref, o_ref):
    s = jax.lax.dot_general(
        x_ref[...],
        mf_ref[...],
        (((1,), (0,)), ((), ())),
        preferred_element_type=jnp.float32,
    )
    idx = jnp.clip(s, 0.0, float(VOCAB - 1)).astype(jnp.int32)
    o_ref[...] = idx.reshape(TCB)


_tc_index = pl.pallas_call(
    _tc_index_body,
    grid=(B // TCB,),
    in_specs=[
        pl.BlockSpec((TCB, F), lambda i: (i, 0)),
        pl.BlockSpec((F, 1), lambda i: (0, 0)),
    ],
    out_specs=pl.BlockSpec((TCB,), lambda i: (i,)),
    out_shape=jax.ShapeDtypeStruct((B,), jnp.int32),
)


@functools.partial(
    pl.kernel,
    out_type=jax.ShapeDtypeStruct((B,), jnp.float32),
    mesh=_mesh,
    compiler_params=pltpu.CompilerParams(needs_layout_passes=False),
    scratch_types=[
        pltpu.VMEM_SHARED((VOCAB,), jnp.float32),
        pltpu.VMEM((BPW,), jnp.int32),
        pltpu.VMEM((BPW,), jnp.float32),
        pltpu.SemaphoreType.DMA,
        pltpu.SemaphoreType.DMA,
    ],
)
def _sc_gather(idx_hbm, fit_hbm, out_hbm, fit_s, idx_v, val_v, si, sg):
    cid = lax.axis_index("c")
    sid = lax.axis_index("s")
    base = (sid * NC + cid) * BPW

    idx_cp = pltpu.async_copy(idx_hbm.at[pl.ds(base, BPW)], idx_v, si)

    @pl.when(sid == 0)
    def _stage_table():
        pltpu.sync_copy(fit_hbm, fit_s)

    plsc.subcore_barrier()
    idx_cp.wait()

    gathers = [
        pltpu.async_copy(
            fit_s.at[idx_v.at[pl.ds(t * GCHUNK, GCHUNK)]],
            val_v.at[pl.ds(t * GCHUNK, GCHUNK)],
            sg,
        )
        for t in range(NGATHER)
    ]
    for g in gathers:
        g.wait()
    pltpu.sync_copy(val_v, out_hbm.at[pl.ds(base, BPW)])


def kernel(x, fitnesses, mult_factor):
    idx = _tc_index(x.reshape(B, F), mult_factor.reshape(F, 1))
    return _sc_gather(idx, fitnesses.reshape(VOCAB))


# speedup vs baseline: 3.2068x; 1.0112x over previous
import functools

import jax
import jax.numpy as jnp
from jax import lax
from jax.experimental import pallas as pl
from jax.experimental.pallas import tpu as pltpu
from jax.experimental.pallas import tpu_sc as plsc

SEQ = 4
NAA = 20
VOCAB = NAA**SEQ
B = 16384
NC, NS, L = 2, 16, 16
NW = NC * NS
BPW = B // NW
GCHUNK = 128
NGATHER = BPW // GCHUNK
TCB = 4096

_mesh = plsc.VectorSubcoreMesh(
    core_axis_name="c", subcore_axis_name="s", num_cores=NC, num_subcores=NS
)


F = SEQ * NAA


def _tc_index_body(x_ref, mf_ref, o_ref):
    s = jax.lax.dot_general(
        x_ref[...],
        mf_ref[...],
        (((1,), (0,)), ((), ())),
        preferred_element_type=jnp.float32,
    )
    idx = jnp.clip(s, 0.0, float(VOCAB - 1)).astype(jnp.int32)
    o_ref[...] = idx.reshape(TCB)


_tc_index = pl.pallas_call(
    _tc_index_body,
    grid=(B // TCB,),
    in_specs=[
        pl.BlockSpec((TCB, F), lambda i: (i, 0)),
        pl.BlockSpec((F, 1), lambda i: (0, 0)),
    ],
    out_specs=pl.BlockSpec((TCB,), lambda i: (i,)),
    out_shape=jax.ShapeDtypeStruct((B,), jnp.int32),
)


@functools.partial(
    pl.kernel,
    out_type=jax.ShapeDtypeStruct((B,), jnp.float32),
    mesh=_mesh,
    compiler_params=pltpu.CompilerParams(needs_layout_passes=False),
    scratch_types=[
        pltpu.VMEM_SHARED((VOCAB,), jnp.float32),
        pltpu.VMEM((BPW,), jnp.int32),
        pltpu.VMEM((BPW,), jnp.float32),
        pltpu.SemaphoreType.DMA,
        pltpu.SemaphoreType.DMA,
    ],
)
def _sc_gather(idx_hbm, fit_hbm, out_hbm, fit_s, idx_v, val_v, si, sg):
    cid = lax.axis_index("c")
    sid = lax.axis_index("s")
    base = (sid * NC + cid) * BPW

    idx_cp = pltpu.async_copy(idx_hbm.at[pl.ds(base, BPW)], idx_v, si)

    @pl.when(sid == 0)
    def _stage_table():
        pltpu.sync_copy(fit_hbm, fit_s)

    plsc.subcore_barrier()
    idx_cp.wait()

    gathers = [
        pltpu.async_copy(
            fit_s.at[idx_v.at[pl.ds(t * GCHUNK, GCHUNK)]],
            val_v.at[pl.ds(t * GCHUNK, GCHUNK)],
            sg,
        )
        for t in range(NGATHER)
    ]
    for g in gathers:
        g.wait()
    pltpu.sync_copy(val_v, out_hbm.at[pl.ds(base, BPW)])


def kernel(x, fitnesses, mult_factor):
    idx = _tc_index(x.reshape(B, F), mult_factor.reshape(F, 1))
    return _sc_gather(idx, fitnesses.reshape(VOCAB))
